# baseline (speedup 1.0000x reference)
import functools

import jax
import jax.numpy as jnp
from jax import lax
from jax.experimental import pallas as pl
from jax.experimental.pallas import tpu as pltpu

_ROWS = 8192
_FEAT = 256
_NCORES = 2
_BLOCK = 2048


def _inner(src, dst):
    dst[...] = src[...]


def _core_body(src_hbm, dst_hbm):
    pipeline = pltpu.emit_pipeline(
        _inner,
        grid=(_ROWS // _BLOCK,),
        in_specs=[pl.BlockSpec((_BLOCK, _FEAT), lambda i: (i, 0))],
        out_specs=[pl.BlockSpec((_BLOCK, _FEAT), lambda i: (i, 0))],
        core_axis_name="x",
        dimension_semantics=(pltpu.PARALLEL,),
    )
    pipeline(src_hbm, dst_hbm)


def kernel(prototypes):
    mesh = pltpu.create_tensorcore_mesh("x", num_cores=_NCORES)
    k = functools.partial(
        pl.kernel,
        mesh=mesh,
        out_type=jax.ShapeDtypeStruct((_ROWS, _FEAT), jnp.float32),
    )(_core_body)
    return k(prototypes)


# speedup vs baseline: 1.0290x; 1.0290x over previous
import jax
import jax.numpy as jnp
from jax.experimental import pallas as pl
from jax.experimental.pallas import tpu as pltpu


_BLOCK_ROWS = 2048


def _copy_kernel(src_ref, dst_ref):
    dst_ref[...] = src_ref[...]


def kernel(prototypes):
    rows, feat = prototypes.shape
    return pl.pallas_call(
        _copy_kernel,
        out_shape=jax.ShapeDtypeStruct(prototypes.shape, prototypes.dtype),
        grid=(rows // _BLOCK_ROWS,),
        in_specs=[pl.BlockSpec((_BLOCK_ROWS, feat), lambda i: (i, 0))],
        out_specs=pl.BlockSpec((_BLOCK_ROWS, feat), lambda i: (i, 0)),
        compiler_params=pltpu.CompilerParams(dimension_semantics=("parallel",)),
    )(prototypes)


# speedup vs baseline: 1.2661x; 1.2304x over previous
import jax
import jax.numpy as jnp
from jax.experimental import pallas as pl
from jax.experimental.pallas import tpu as pltpu


_BLOCK_ROWS = 4096


def _copy_kernel(src_ref, dst_ref):
    dst_ref[...] = src_ref[...]


def kernel(prototypes):
    rows, feat = prototypes.shape
    return pl.pallas_call(
        _copy_kernel,
        out_shape=jax.ShapeDtypeStruct(prototypes.shape, prototypes.dtype),
        grid=(rows // _BLOCK_ROWS,),
        in_specs=[pl.BlockSpec((_BLOCK_ROWS, feat), lambda i: (i, 0))],
        out_specs=pl.BlockSpec((_BLOCK_ROWS, feat), lambda i: (i, 0)),
    )(prototypes)
